# Initial kernel scaffold; baseline (speedup 1.0000x reference)
import jax
import jax.numpy as jnp
from jax.experimental import pallas as pl


def kernel(t_feat, v_feat, t_weight, t_bias, v_weight, v_bias):
    raise NotImplementedError("write your pallas kernel here")


# speedup vs baseline: 4.3397x; 4.3397x over previous
import functools

import numpy as np
import jax
import jax.numpy as jnp
from jax import lax
from jax.experimental import pallas as pl
from jax.experimental.pallas import tpu as pltpu
from jax.experimental.pallas import tpu_sc as plsc

N_ITEMS = 10000
F_DIM = 512
LAT_HALF = 64
KNN_K = 5
NP = 10240
RB = 1024
CB = 1024
N_WORKERS = 32
RPW = NP // N_WORKERS
CHUNK = 64
DEG = KNN_K * 2

_RS = np.float32(np.float32(5.0 + 1e-7) ** np.float32(-0.5))
_VAL = np.float32(_RS * _RS)
W_V = float(np.float32(0.2) * _VAL)
W_T = float(np.float32(0.8) * _VAL)


def _proj_body(t_ref, v_ref, tw_ref, vw_ref, tb_ref, vb_ref,
               pv_ref, pt_ref, nt_ref, nv_ref):
    t = t_ref[...]
    v = v_ref[...]
    pt_ref[...] = jnp.dot(t, tw_ref[...], preferred_element_type=jnp.float32,
                          precision=lax.Precision.HIGHEST) + tb_ref[...]
    pv_ref[...] = jnp.dot(v, vw_ref[...], preferred_element_type=jnp.float32,
                          precision=lax.Precision.HIGHEST) + vb_ref[...]
    nt_ref[...] = t / jnp.sqrt(jnp.sum(t * t, axis=1, keepdims=True))
    nv_ref[...] = v / jnp.sqrt(jnp.sum(v * v, axis=1, keepdims=True))


def _projections(t_feat, v_feat, t_weight, t_bias, v_weight, v_bias,
                 interpret=False):
    nblk = 10
    rb = N_ITEMS // nblk
    return pl.pallas_call(
        _proj_body,
        grid=(nblk,),
        in_specs=[
            pl.BlockSpec((rb, F_DIM), lambda i: (i, 0)),
            pl.BlockSpec((rb, F_DIM), lambda i: (i, 0)),
            pl.BlockSpec((F_DIM, LAT_HALF), lambda i: (0, 0)),
            pl.BlockSpec((F_DIM, LAT_HALF), lambda i: (0, 0)),
            pl.BlockSpec((rb, LAT_HALF), lambda i: (i, 0)),
            pl.BlockSpec((rb, LAT_HALF), lambda i: (i, 0)),
        ],
        out_specs=[
            pl.BlockSpec((rb, LAT_HALF), lambda i: (i, 0)),
            pl.BlockSpec((rb, LAT_HALF), lambda i: (i, 0)),
            pl.BlockSpec((rb, F_DIM), lambda i: (i, 0)),
            pl.BlockSpec((rb, F_DIM), lambda i: (i, 0)),
        ],
        out_shape=[
            jax.ShapeDtypeStruct((N_ITEMS, LAT_HALF), jnp.float32),
            jax.ShapeDtypeStruct((N_ITEMS, LAT_HALF), jnp.float32),
            jax.ShapeDtypeStruct((N_ITEMS, F_DIM), jnp.float32),
            jax.ShapeDtypeStruct((N_ITEMS, F_DIM), jnp.float32),
        ],
        interpret=interpret,
    )(t_feat, v_feat, t_weight, v_weight, t_bias, v_bias)


def _topk_body(ncols, a_ref, b_ref, out_ref, rv_ref, ri_ref):
    j = pl.program_id(1)

    @pl.when(j == 0)
    def _():
        rv_ref[...] = jnp.full((RB, 128), -jnp.inf, jnp.float32)
        ri_ref[...] = jnp.zeros((RB, 128), jnp.int32)

    s = lax.dot_general(a_ref[...], b_ref[...], (((1,), (1,)), ((), ())),
                        preferred_element_type=jnp.float32)
    gcol = j * CB + lax.broadcasted_iota(jnp.int32, (RB, CB), 1)
    s = jnp.where(gcol >= N_ITEMS, -jnp.inf, s)

    big = jnp.int32(2 ** 30)
    ms, ps = [], []
    for _ in range(KNN_K):
        m = jnp.max(s, axis=1)
        pos = jnp.min(jnp.where(s == m[:, None], gcol, big), axis=1)
        s = jnp.where(gcol == pos[:, None], -jnp.inf, s)
        ms.append(m)
        ps.append(pos)

    rv = rv_ref[...]
    ri = ri_ref[...]
    li = lax.broadcasted_iota(jnp.int32, (RB, 128), 1)
    for k in range(KNN_K):
        rv = jnp.where(li == 8 + k, ms[k][:, None], rv)
        ri = jnp.where(li == 8 + k, ps[k][:, None], ri)
    nrv = jnp.full((RB, 128), -jnp.inf, jnp.float32)
    nri = jnp.zeros((RB, 128), jnp.int32)
    for k in range(KNN_K):
        m = jnp.max(rv, axis=1)
        pos = jnp.min(jnp.where(rv == m[:, None], li, big), axis=1)
        sel = li == pos[:, None]
        iv = jnp.sum(jnp.where(sel, ri, 0), axis=1)
        nrv = jnp.where(li == k, m[:, None], nrv)
        nri = jnp.where(li == k, iv[:, None], nri)
        rv = jnp.where(sel, -jnp.inf, rv)
    rv_ref[...] = nrv
    ri_ref[...] = nri

    @pl.when(j == ncols - 1)
    def _():
        out_ref[...] = nri


def _knn_topk(nrm_pad, interpret=False):
    grid = (NP // RB, NP // CB)
    out = pl.pallas_call(
        functools.partial(_topk_body, grid[1]),
        grid=grid,
        in_specs=[pl.BlockSpec((RB, F_DIM), lambda i, j: (i, 0)),
                  pl.BlockSpec((CB, F_DIM), lambda i, j: (j, 0))],
        out_specs=pl.BlockSpec((RB, 128), lambda i, j: (i, 0)),
        out_shape=jax.ShapeDtypeStruct((NP, 128), jnp.int32),
        scratch_shapes=[pltpu.VMEM((RB, 128), jnp.float32),
                        pltpu.VMEM((RB, 128), jnp.int32)],
        compiler_params=pltpu.CompilerParams(
            dimension_semantics=("arbitrary", "arbitrary")),
        interpret=interpret,
    )(nrm_pad, nrm_pad)
    return out[:N_ITEMS, :KNN_K]


@functools.lru_cache(maxsize=None)
def _make_gcn_layer(add_base):
    mesh = plsc.VectorSubcoreMesh(core_axis_name="c", subcore_axis_name="s")
    scratch = [
        pltpu.VMEM((8, 128), jnp.int32),
        pltpu.VMEM((CHUNK * DEG, 128), jnp.float32),
        pltpu.VMEM((CHUNK, 128), jnp.float32),
        pltpu.VMEM((CHUNK, 128), jnp.float32),
        pltpu.SemaphoreType.DMA,
    ]

    def body(*refs):
        if add_base:
            h_hbm, idx_hbm, rep_hbm, out_hbm, idxb, gat, obuf, bbuf, sem = refs
        else:
            h_hbm, idx_hbm, out_hbm, idxb, gat, obuf, bbuf, sem = refs
        wid = lax.axis_index("s") * 2 + lax.axis_index("c")
        row0 = wid * RPW

        def chunk_body(ci, carry):
            base = row0 + ci * CHUNK
            idx_off = (base // CHUNK) * 8
            pltpu.sync_copy(idx_hbm.at[pl.ds(idx_off, 8)], idxb)
            copies = []
            for jj in range(KNN_K):
                copies.append(pltpu.async_copy(
                    h_hbm.at[idxb.at[jj]],
                    gat.at[pl.ds(jj * 128, 128)], sem))
            for cp in copies:
                cp.wait()
            if add_base:
                pltpu.sync_copy(rep_hbm.at[pl.ds(base, CHUNK)], bbuf)

            def row_body(r, c2):
                for g in range(8):
                    sl = pl.ds(g * 16, 16)
                    av = (gat[r * DEG + 0, sl] + gat[r * DEG + 1, sl]
                          + gat[r * DEG + 2, sl] + gat[r * DEG + 3, sl]
                          + gat[r * DEG + 4, sl])
                    at_ = (gat[r * DEG + 5, sl] + gat[r * DEG + 6, sl]
                           + gat[r * DEG + 7, sl] + gat[r * DEG + 8, sl]
                           + gat[r * DEG + 9, sl])
                    val = av * W_V + at_ * W_T
                    if add_base:
                        val = val + bbuf[r, sl]
                    obuf[r, sl] = val
                return c2

            lax.fori_loop(0, CHUNK, row_body, 0)
            pltpu.sync_copy(obuf, out_hbm.at[pl.ds(base, CHUNK)])
            return carry

        lax.fori_loop(0, RPW // CHUNK, chunk_body, 0)

    return pl.kernel(
        body,
        out_type=jax.ShapeDtypeStruct((NP, 128), jnp.float32),
        mesh=mesh,
        scratch_types=scratch,
    )


def kernel(t_feat, v_feat, t_weight, t_bias, v_weight, v_bias):
    pv, pt, nt, nv = _projections(t_feat, v_feat, t_weight, t_bias,
                                  v_weight, v_bias)
    item_rep = jnp.concatenate([pv, pt], axis=1)

    pad = NP - N_ITEMS
    knn_v = _knn_topk(jnp.pad(nv, ((0, pad), (0, 0))))
    knn_t = _knn_topk(jnp.pad(nt, ((0, pad), (0, 0))))

    idx = jnp.concatenate([knn_v, knn_t], axis=1)
    idx = jnp.pad(idx, ((0, pad), (0, 0))).reshape(NP // CHUNK, CHUNK * DEG)
    idx = jnp.pad(idx, ((0, 0), (0, 8 * 128 - CHUNK * DEG)))
    idx = idx.reshape(NP // CHUNK * 8, 128)
    rep_pad = jnp.pad(item_rep, ((0, pad), (0, 0)))

    h1 = _make_gcn_layer(add_base=False)(rep_pad, idx)
    out = _make_gcn_layer(add_base=True)(h1, idx, rep_pad)
    return out[:N_ITEMS]
